# Initial kernel scaffold; baseline (speedup 1.0000x reference)
import jax
import jax.numpy as jnp
from jax.experimental import pallas as pl


def kernel(embeds, edge_index, qTrans, kTrans, vTrans):
    raise NotImplementedError("write your pallas kernel here")


# speedup vs baseline: 2.4218x; 2.4218x over previous
import jax
import jax.numpy as jnp
from jax import lax
from jax.experimental import pallas as pl
from jax.experimental.pallas import tpu as pltpu
from jax.experimental.pallas import tpu_sc as plsc

N = 10000
E = 320000
D = 128
H = 4
HD = D // H

NC = 2
NS = 16
NW = NC * NS
L = 16

EPW = E // NW
C = 80
NCHUNK = EPW // C

NPAD = 10240
STRIPE = NPAD // NS

_EPS = 1e-8

_SC_PARAMS = pltpu.CompilerParams(
    use_tc_tiling_on_sc=False, needs_layout_passes=False)


def _mesh():
    return plsc.VectorSubcoreMesh(core_axis_name="c", subcore_axis_name="s",
                                  num_cores=NC, num_subcores=NS)


def _qkv_body(emb_ref, q_ref, k_ref, v_ref, qn_ref, kn_ref, vn_ref):
    e = emb_ref[...]
    qn_ref[...] = jnp.dot(e, q_ref[...], preferred_element_type=jnp.float32)
    kn_ref[...] = jnp.dot(e, k_ref[...], preferred_element_type=jnp.float32)
    vn_ref[...] = jnp.dot(e, v_ref[...], preferred_element_type=jnp.float32)


def _qkv(emb_pad, qT, kT, vT):
    blk = 1024
    nd = jax.ShapeDtypeStruct((NPAD, D), jnp.float32)
    wspec = pl.BlockSpec((D, D), lambda i: (0, 0))
    nspec = pl.BlockSpec((blk, D), lambda i: (i, 0))
    return pl.pallas_call(
        _qkv_body,
        grid=(NPAD // blk,),
        in_specs=[nspec, wspec, wspec, wspec],
        out_specs=[nspec, nspec, nspec],
        out_shape=[nd, nd, nd],
    )(emb_pad, qT, kT, vT)


def _edge_body(qn, kn, rows, cols, expatt, normp,
               rows_v, cols_v, qe, ke, vatt, eaf, bflat, norm_sh, zbuf,
               sem1, sem2):
    cid = lax.axis_index("c")
    sid = lax.axis_index("s")
    wid = cid * NS + sid

    zero16 = jnp.zeros((L,), jnp.float32)

    @pl.loop(0, STRIPE)
    def _zero(r):
        zbuf[r, pl.ds(0, L)] = zero16

    pltpu.sync_copy(zbuf, norm_sh.at[pl.ds(sid * STRIPE, STRIPE)])
    plsc.subcore_barrier()

    iota = lax.iota(jnp.int32, L)

    @pl.loop(0, NCHUNK)
    def _chunk(i):
        base = wid * EPW + i * C
        pltpu.sync_copy(rows.at[pl.ds(base, C)], rows_v)
        pltpu.sync_copy(cols.at[pl.ds(base, C)], cols_v)
        cp1 = pltpu.async_copy(qn.at[rows_v], qe, sem1)
        cp2 = pltpu.async_copy(kn.at[cols_v], ke, sem2)
        cp1.wait()
        cp2.wait()

        @pl.loop(0, C)
        def _edge(e):
            q = [qe[e, pl.ds(L * j, L)] for j in range(D // L)]
            k = [ke[e, pl.ds(L * j, L)] for j in range(D // L)]
            a = []
            for h in range(H):
                m = q[2 * h] * k[2 * h] + q[2 * h + 1] * k[2 * h + 1]
                s = jnp.sum(m)
                s = jnp.minimum(jnp.maximum(s, -10.0), 10.0)
                a.append(jnp.exp(jnp.full((L,), s, jnp.float32)))
            comb = jnp.where(iota == 0, a[0],
                             jnp.where(iota == 1, a[1],
                                       jnp.where(iota == 2, a[2], a[3])))
            vatt[e, pl.ds(0, L)] = comb

        for g in range(C // L):
            ridx = iota + g * L
            for h in range(H):
                v = plsc.load_gather(vatt, [ridx, jnp.full((L,), h, jnp.int32)])
                plsc.store_scatter(eaf, [ridx * H + h], v)

        pltpu.sync_copy(vatt, norm_sh.at[rows_v], add=True)
        pltpu.sync_copy(eaf, expatt.at[pl.ds(base * H, C * H)])

    plsc.subcore_barrier()
    pltpu.sync_copy(norm_sh.at[pl.ds(sid * STRIPE, STRIPE)], zbuf)

    @pl.loop(0, STRIPE)
    def _repack(r):
        bflat[pl.ds(r * L, L)] = zbuf[r, pl.ds(0, L)]

    out0 = cid * (NPAD * L) + sid * (STRIPE * L)
    pltpu.sync_copy(bflat, normp.at[pl.ds(out0, STRIPE * L)])


def _edge_pass(qn, kn, rows, cols):
    f = pl.kernel(
        _edge_body,
        out_type=[
            jax.ShapeDtypeStruct((E * H,), jnp.float32),
            jax.ShapeDtypeStruct((NC * NPAD * L,), jnp.float32),
        ],
        mesh=_mesh(),
        scratch_types=[
            pltpu.VMEM((C,), jnp.int32),
            pltpu.VMEM((C,), jnp.int32),
            pltpu.VMEM((C, D), jnp.float32),
            pltpu.VMEM((C, D), jnp.float32),
            pltpu.VMEM((C, L), jnp.float32),
            pltpu.VMEM((C * H,), jnp.float32),
            pltpu.VMEM((STRIPE * L,), jnp.float32),
            pltpu.VMEM_SHARED((NPAD, L), jnp.float32),
            pltpu.VMEM((STRIPE, L), jnp.float32),
            pltpu.SemaphoreType.DMA,
            pltpu.SemaphoreType.DMA,
        ],
        compiler_params=_SC_PARAMS,
    )
    return f(qn, kn, rows, cols)


def _recip_body(n_ref, r_ref):
    half = NPAD * L // D
    r_ref[...] = 1.0 / (n_ref[:half] + n_ref[half:] + _EPS)


def _recip(normp):
    n2 = normp.reshape(NC * NPAD * L // D, D)
    out = pl.pallas_call(
        _recip_body,
        out_shape=jax.ShapeDtypeStruct((NPAD * L // D, D), jnp.float32),
    )(n2)
    return out.reshape(NPAD * L)


def _aggr_body(vn, rows, cols, expatt, recip, att, numerp,
               rows_v, cols_v, ve, eab, idx2, rcp, attb, val,
               numer_sh, zbuf, sem1, sem2):
    cid = lax.axis_index("c")
    sid = lax.axis_index("s")
    wid = cid * NS + sid

    zero16 = jnp.zeros((L,), jnp.float32)

    @pl.loop(0, STRIPE // 5)
    def _zero(r):
        for j in range(D // L):
            zbuf[r, pl.ds(L * j, L)] = zero16

    for p in range(5):
        pltpu.sync_copy(
            zbuf, numer_sh.at[pl.ds(sid * STRIPE + p * (STRIPE // 5),
                                    STRIPE // 5)])
    plsc.subcore_barrier()

    iota = lax.iota(jnp.int32, L)

    @pl.loop(0, NCHUNK)
    def _chunk(i):
        base = wid * EPW + i * C
        pltpu.sync_copy(rows.at[pl.ds(base, C)], rows_v)
        pltpu.sync_copy(cols.at[pl.ds(base, C)], cols_v)
        cp1 = pltpu.async_copy(vn.at[cols_v], ve, sem1)
        pltpu.sync_copy(expatt.at[pl.ds(base * H, C * H)], eab)

        for j in range(H):
            for u in range(C // L):
                t = 80 * j + 16 * u + iota
                e_idx = lax.shift_right_logical(t, 2)
                hh = lax.bitwise_and(t, 3)
                rv = plsc.load_gather(rows_v, [e_idx])
                idx2[j, pl.ds(L * u, L)] = rv * L + hh

        for j in range(H):
            pltpu.async_copy(recip.at[idx2.at[j]],
                             rcp.at[pl.ds(C * j, C)], sem2).wait()

        for u in range(C * H // L):
            attb[pl.ds(L * u, L)] = (eab[pl.ds(L * u, L)]
                                     * rcp[pl.ds(L * u, L)])

        cp1.wait()

        @pl.loop(0, C)
        def _edge(e):
            for h in range(H):
                sp = plsc.load_gather(attb, [jnp.full((L,), e * H + h,
                                                      jnp.int32)])
                o = 2 * h * L
                val[e, pl.ds(o, L)] = sp * ve[e, pl.ds(o, L)]
                val[e, pl.ds(o + L, L)] = sp * ve[e, pl.ds(o + L, L)]

        pltpu.sync_copy(val, numer_sh.at[rows_v], add=True)
        pltpu.sync_copy(attb, att.at[pl.ds(base * H, C * H)])

    plsc.subcore_barrier()
    sl = pl.ds(sid * STRIPE, STRIPE)
    pltpu.sync_copy(numer_sh.at[sl], numerp.at[cid, sl])


def _aggr_pass(vn, rows, cols, expatt, recip):
    f = pl.kernel(
        _aggr_body,
        out_type=[
            jax.ShapeDtypeStruct((E * H,), jnp.float32),
            jax.ShapeDtypeStruct((NC, NPAD, D), jnp.float32),
        ],
        mesh=_mesh(),
        scratch_types=[
            pltpu.VMEM((C,), jnp.int32),
            pltpu.VMEM((C,), jnp.int32),
            pltpu.VMEM((C, D), jnp.float32),
            pltpu.VMEM((C * H,), jnp.float32),
            pltpu.VMEM((H, C), jnp.int32),
            pltpu.VMEM((C * H,), jnp.float32),
            pltpu.VMEM((C * H,), jnp.float32),
            pltpu.VMEM((C, D), jnp.float32),
            pltpu.VMEM_SHARED((NPAD, D), jnp.float32),
            pltpu.VMEM((STRIPE // 5, D), jnp.float32),
            pltpu.SemaphoreType.DMA,
            pltpu.SemaphoreType.DMA,
        ],
        compiler_params=_SC_PARAMS,
    )
    return f(vn, rows, cols, expatt, recip)


def _sum_body(n_ref, o_ref):
    o_ref[...] = n_ref[0] + n_ref[1]


def _psum(numerp):
    blk = 2048
    return pl.pallas_call(
        _sum_body,
        grid=(NPAD // blk,),
        in_specs=[pl.BlockSpec((NC, blk, D), lambda i: (0, i, 0))],
        out_specs=pl.BlockSpec((blk, D), lambda i: (i, 0)),
        out_shape=jax.ShapeDtypeStruct((NPAD, D), jnp.float32),
    )(numerp)


@jax.jit
def kernel(embeds, edge_index, qTrans, kTrans, vTrans):
    rows = edge_index[0, :].astype(jnp.int32)
    cols = edge_index[1, :].astype(jnp.int32)
    emb_pad = jnp.pad(embeds, ((0, NPAD - N), (0, 0)))

    qn, kn, vn = _qkv(emb_pad, qTrans, kTrans, vTrans)
    expatt, normp = _edge_pass(qn, kn, rows, cols)
    recip = _recip(normp)
    attf, numerp = _aggr_pass(vn, rows, cols, expatt, recip)
    res = _psum(numerp)[:N]
    att = attf.reshape(E, H)
    return res, att


# speedup vs baseline: 4.0986x; 1.6924x over previous
import jax
import jax.numpy as jnp
from jax import lax
from jax.experimental import pallas as pl
from jax.experimental.pallas import tpu as pltpu
from jax.experimental.pallas import tpu_sc as plsc

N = 10000
E = 320000
D = 128
H = 4
HD = D // H

NC = 2
NS = 16
NW = NC * NS
L = 16

EPW = E // NW
C = 80
NCHUNK = EPW // C

NPAD = 10240
STRIPE = NPAD // NS

_EPS = 1e-8

_SC_PARAMS = pltpu.CompilerParams(
    use_tc_tiling_on_sc=False, needs_layout_passes=False)


def _mesh():
    return plsc.VectorSubcoreMesh(core_axis_name="c", subcore_axis_name="s",
                                  num_cores=NC, num_subcores=NS)


def _qkv_body(emb_ref, q_ref, k_ref, v_ref, qn_ref, kn_ref, vn_ref):
    e = emb_ref[...]
    qn_ref[...] = jnp.dot(e, q_ref[...], preferred_element_type=jnp.float32)
    kn_ref[...] = jnp.dot(e, k_ref[...], preferred_element_type=jnp.float32)
    vn_ref[...] = jnp.dot(e, v_ref[...], preferred_element_type=jnp.float32)


def _qkv(emb_pad, qT, kT, vT):
    blk = 1024
    nd = jax.ShapeDtypeStruct((NPAD, D), jnp.float32)
    wspec = pl.BlockSpec((D, D), lambda i: (0, 0))
    nspec = pl.BlockSpec((blk, D), lambda i: (i, 0))
    return pl.pallas_call(
        _qkv_body,
        grid=(NPAD // blk,),
        in_specs=[nspec, wspec, wspec, wspec],
        out_specs=[nspec, nspec, nspec],
        out_shape=[nd, nd, nd],
    )(emb_pad, qT, kT, vT)


def _edge_body(qn, kn, rows, cols, expatt, normp,
               rows_v, cols_v, qe, ke, vatt, eaf, bflat, norm_sh, zbuf,
               sem1, sem2):
    cid = lax.axis_index("c")
    sid = lax.axis_index("s")
    wid = cid * NS + sid

    zero16 = jnp.zeros((L,), jnp.float32)

    @pl.loop(0, STRIPE)
    def _zero(r):
        zbuf[r, pl.ds(0, L)] = zero16

    pltpu.sync_copy(zbuf, norm_sh.at[pl.ds(sid * STRIPE, STRIPE)])
    plsc.subcore_barrier()

    iota = lax.iota(jnp.int32, L)

    @pl.loop(0, NCHUNK)
    def _chunk(i):
        base = wid * EPW + i * C
        pltpu.sync_copy(rows.at[pl.ds(base, C)], rows_v)
        pltpu.sync_copy(cols.at[pl.ds(base, C)], cols_v)
        cp1 = pltpu.async_copy(qn.at[rows_v], qe, sem1)
        cp2 = pltpu.async_copy(kn.at[cols_v], ke, sem2)
        cp1.wait()
        cp2.wait()

        @plsc.parallel_loop(0, C, unroll=4)
        def _edge(e):
            q = [qe[e, pl.ds(L * j, L)] for j in range(D // L)]
            k = [ke[e, pl.ds(L * j, L)] for j in range(D // L)]
            a = []
            for h in range(H):
                m = q[2 * h] * k[2 * h] + q[2 * h + 1] * k[2 * h + 1]
                s = jnp.sum(m)
                s = jnp.minimum(jnp.maximum(s, -10.0), 10.0)
                a.append(jnp.exp(jnp.full((L,), s, jnp.float32)))
            comb = jnp.where(iota == 0, a[0],
                             jnp.where(iota == 1, a[1],
                                       jnp.where(iota == 2, a[2], a[3])))
            vatt[e, pl.ds(0, L)] = comb

        for g in range(C // L):
            ridx = iota + g * L
            for h in range(H):
                v = plsc.load_gather(vatt, [ridx, jnp.full((L,), h, jnp.int32)])
                plsc.store_scatter(eaf, [ridx * H + h], v)

        pltpu.sync_copy(vatt, norm_sh.at[rows_v], add=True)
        pltpu.sync_copy(eaf, expatt.at[pl.ds(base * H, C * H)])

    plsc.subcore_barrier()
    pltpu.sync_copy(norm_sh.at[pl.ds(sid * STRIPE, STRIPE)], zbuf)
    rsub = lax.shift_right_logical(iota, 2)
    csub = lax.bitwise_and(iota, 3)

    @pl.loop(0, STRIPE, step=4)
    def _repack(r):
        v = plsc.load_gather(zbuf, [r + rsub, csub])
        bflat[pl.ds(r * H, L)] = v

    out0 = cid * (NPAD * H) + sid * (STRIPE * H)
    pltpu.sync_copy(bflat, normp.at[pl.ds(out0, STRIPE * H)])


def _edge_pass(qn, kn, rows, cols):
    f = pl.kernel(
        _edge_body,
        out_type=[
            jax.ShapeDtypeStruct((E * H,), jnp.float32),
            jax.ShapeDtypeStruct((NC * NPAD * H,), jnp.float32),
        ],
        mesh=_mesh(),
        scratch_types=[
            pltpu.VMEM((C,), jnp.int32),
            pltpu.VMEM((C,), jnp.int32),
            pltpu.VMEM((C, D), jnp.float32),
            pltpu.VMEM((C, D), jnp.float32),
            pltpu.VMEM((C, L), jnp.float32),
            pltpu.VMEM((C * H,), jnp.float32),
            pltpu.VMEM((STRIPE * H,), jnp.float32),
            pltpu.VMEM_SHARED((NPAD, L), jnp.float32),
            pltpu.VMEM((STRIPE, L), jnp.float32),
            pltpu.SemaphoreType.DMA,
            pltpu.SemaphoreType.DMA,
        ],
        compiler_params=_SC_PARAMS,
    )
    return f(qn, kn, rows, cols)


def _recip_body(n_ref, r_ref):
    half = NPAD * H // D
    r_ref[...] = 1.0 / (n_ref[:half] + n_ref[half:] + _EPS)


def _recip(normp):
    n2 = normp.reshape(NC * NPAD * H // D, D)
    out = pl.pallas_call(
        _recip_body,
        out_shape=jax.ShapeDtypeStruct((NPAD * H // D, D), jnp.float32),
    )(n2)
    return out.reshape(NPAD * H)


def _aggr_body(vn, rows, cols, expatt, recip, att, numerp,
               rows_v, cols_v, ve, eab, idx2, rcp, attb, val,
               numer_sh, zbuf, sem1, sem2):
    cid = lax.axis_index("c")
    sid = lax.axis_index("s")
    wid = cid * NS + sid

    zero16 = jnp.zeros((L,), jnp.float32)

    @pl.loop(0, STRIPE // 5)
    def _zero(r):
        for j in range(D // L):
            zbuf[r, pl.ds(L * j, L)] = zero16

    for p in range(5):
        pltpu.sync_copy(
            zbuf, numer_sh.at[pl.ds(sid * STRIPE + p * (STRIPE // 5),
                                    STRIPE // 5)])
    plsc.subcore_barrier()

    iota = lax.iota(jnp.int32, L)
    e_sub = lax.shift_right_logical(iota, 2)
    h_sub = lax.bitwise_and(iota, 3)

    @pl.loop(0, NCHUNK)
    def _chunk(i):
        base = wid * EPW + i * C
        pltpu.sync_copy(rows.at[pl.ds(base, C)], rows_v)
        pltpu.sync_copy(cols.at[pl.ds(base, C)], cols_v)
        cp1 = pltpu.async_copy(vn.at[cols_v], ve, sem1)
        pltpu.sync_copy(expatt.at[pl.ds(base * H, C * H)], eab)

        @plsc.parallel_loop(0, C, step=4, unroll=4)
        def _idx16(e0):
            rv = plsc.load_gather(rows_v, [e0 + e_sub])
            j = e0 // (C // H)
            u = e0 % (C // H)
            idx2[j, pl.ds(u * H, L)] = rv * H + h_sub

        cps = [pltpu.async_copy(recip.at[idx2.at[j]],
                                rcp.at[pl.ds(C * j, C)], sem2)
               for j in range(H)]
        for cp in cps:
            cp.wait()

        @plsc.parallel_loop(0, C * H // L, unroll=4)
        def _att16(u):
            attb[pl.ds(L * u, L)] = (eab[pl.ds(L * u, L)]
                                     * rcp[pl.ds(L * u, L)])

        cp1.wait()

        @plsc.parallel_loop(0, C, unroll=4)
        def _edge(e):
            for h in range(H):
                sp = plsc.load_gather(attb, [jnp.full((L,), e * H + h,
                                                      jnp.int32)])
                o = 2 * h * L
                val[e, pl.ds(o, L)] = sp * ve[e, pl.ds(o, L)]
                val[e, pl.ds(o + L, L)] = sp * ve[e, pl.ds(o + L, L)]

        pltpu.sync_copy(val, numer_sh.at[rows_v], add=True)
        pltpu.sync_copy(attb, att.at[pl.ds(base * H, C * H)])

    plsc.subcore_barrier()
    sl = pl.ds(sid * STRIPE, STRIPE)
    pltpu.sync_copy(numer_sh.at[sl], numerp.at[cid, sl])


def _aggr_pass(vn, rows, cols, expatt, recip):
    f = pl.kernel(
        _aggr_body,
        out_type=[
            jax.ShapeDtypeStruct((E * H,), jnp.float32),
            jax.ShapeDtypeStruct((NC, NPAD, D), jnp.float32),
        ],
        mesh=_mesh(),
        scratch_types=[
            pltpu.VMEM((C,), jnp.int32),
            pltpu.VMEM((C,), jnp.int32),
            pltpu.VMEM((C, D), jnp.float32),
            pltpu.VMEM((C * H,), jnp.float32),
            pltpu.VMEM((H, C), jnp.int32),
            pltpu.VMEM((C * H,), jnp.float32),
            pltpu.VMEM((C * H,), jnp.float32),
            pltpu.VMEM((C, D), jnp.float32),
            pltpu.VMEM_SHARED((NPAD, D), jnp.float32),
            pltpu.VMEM((STRIPE // 5, D), jnp.float32),
            pltpu.SemaphoreType.DMA,
            pltpu.SemaphoreType.DMA,
        ],
        compiler_params=_SC_PARAMS,
    )
    return f(vn, rows, cols, expatt, recip)


def _sum_body(n_ref, o_ref):
    o_ref[...] = n_ref[0] + n_ref[1]


def _psum(numerp):
    blk = 2048
    return pl.pallas_call(
        _sum_body,
        grid=(NPAD // blk,),
        in_specs=[pl.BlockSpec((NC, blk, D), lambda i: (0, i, 0))],
        out_specs=pl.BlockSpec((blk, D), lambda i: (i, 0)),
        out_shape=jax.ShapeDtypeStruct((NPAD, D), jnp.float32),
    )(numerp)


@jax.jit
def kernel(embeds, edge_index, qTrans, kTrans, vTrans):
    rows = edge_index[0, :].astype(jnp.int32)
    cols = edge_index[1, :].astype(jnp.int32)
    emb_pad = jnp.pad(embeds, ((0, NPAD - N), (0, 0)))

    qn, kn, vn = _qkv(emb_pad, qTrans, kTrans, vTrans)
    expatt, normp = _edge_pass(qn, kn, rows, cols)
    recip = _recip(normp)
    attf, numerp = _aggr_pass(vn, rows, cols, expatt, recip)
    res = _psum(numerp)[:N]
    att = attf.reshape(E, H)
    return res, att


# speedup vs baseline: 4.7132x; 1.1500x over previous
import jax
import jax.numpy as jnp
from jax import lax
from jax.experimental import pallas as pl
from jax.experimental.pallas import tpu as pltpu
from jax.experimental.pallas import tpu_sc as plsc

N = 10000
E = 320000
D = 128
H = 4
HD = D // H

NC = 2
NS = 16
NW = NC * NS
L = 16

EPW = E // NW
C = 80
NCHUNK = EPW // C

NPAD = 10240
STRIPE = NPAD // NS

_EPS = 1e-8

_SC_PARAMS = pltpu.CompilerParams(
    use_tc_tiling_on_sc=False, needs_layout_passes=False)


def _mesh():
    return plsc.VectorSubcoreMesh(core_axis_name="c", subcore_axis_name="s",
                                  num_cores=NC, num_subcores=NS)


def _qkv_body(emb_ref, q_ref, k_ref, v_ref, qn_ref, kn_ref, vn_ref):
    e = emb_ref[...]
    qn_ref[...] = jnp.dot(e, q_ref[...], preferred_element_type=jnp.float32)
    kn_ref[...] = jnp.dot(e, k_ref[...], preferred_element_type=jnp.float32)
    vn_ref[...] = jnp.dot(e, v_ref[...], preferred_element_type=jnp.float32)


def _qkv(emb_pad, qT, kT, vT):
    blk = 1024
    nd = jax.ShapeDtypeStruct((NPAD, D), jnp.float32)
    wspec = pl.BlockSpec((D, D), lambda i: (0, 0))
    nspec = pl.BlockSpec((blk, D), lambda i: (i, 0))
    return pl.pallas_call(
        _qkv_body,
        grid=(NPAD // blk,),
        in_specs=[nspec, wspec, wspec, wspec],
        out_specs=[nspec, nspec, nspec],
        out_shape=[nd, nd, nd],
    )(emb_pad, qT, kT, vT)


def _edge_body(qn, kn, rows, cols, expatt, normp,
               rv0, cv0, qe0, ke0, rv1, cv1, qe1, ke1,
               vatt, eaf, bflat, norm_sh, zbuf,
               sq0, sk0, sq1, sk1):
    cid = lax.axis_index("c")
    sid = lax.axis_index("s")
    wid = cid * NS + sid
    bufs = ((rv0, cv0, qe0, ke0, sq0, sk0), (rv1, cv1, qe1, ke1, sq1, sk1))

    zero16 = jnp.zeros((L,), jnp.float32)

    @pl.loop(0, STRIPE)
    def _zero(r):
        zbuf[r, pl.ds(0, L)] = zero16

    pltpu.sync_copy(zbuf, norm_sh.at[pl.ds(sid * STRIPE, STRIPE)])
    plsc.subcore_barrier()

    iota = lax.iota(jnp.int32, L)

    def start(b, c):
        rv, cv, qe, ke, sq, sk = bufs[b]
        base = wid * EPW + c * C
        pltpu.sync_copy(rows.at[pl.ds(base, C)], rv)
        pltpu.sync_copy(cols.at[pl.ds(base, C)], cv)
        pltpu.async_copy(qn.at[rv], qe, sq)
        pltpu.async_copy(kn.at[cv], ke, sk)

    def finish(b, c):
        rv, cv, qe, ke, sq, sk = bufs[b]
        pltpu.make_async_copy(qn.at[rv], qe, sq).wait()
        pltpu.make_async_copy(kn.at[cv], ke, sk).wait()

        @plsc.parallel_loop(0, C, unroll=4)
        def _edge(e):
            q = [qe[e, pl.ds(L * j, L)] for j in range(D // L)]
            k = [ke[e, pl.ds(L * j, L)] for j in range(D // L)]
            a = []
            for h in range(H):
                m = q[2 * h] * k[2 * h] + q[2 * h + 1] * k[2 * h + 1]
                s = jnp.sum(m)
                s = jnp.minimum(jnp.maximum(s, -10.0), 10.0)
                a.append(jnp.exp(jnp.full((L,), s, jnp.float32)))
            comb = jnp.where(iota == 0, a[0],
                             jnp.where(iota == 1, a[1],
                                       jnp.where(iota == 2, a[2], a[3])))
            vatt[e, pl.ds(0, L)] = comb

        for g in range(C // L):
            ridx = iota + g * L
            for h in range(H):
                v = plsc.load_gather(vatt, [ridx, jnp.full((L,), h, jnp.int32)])
                plsc.store_scatter(eaf, [ridx * H + h], v)

        pltpu.sync_copy(vatt, norm_sh.at[rv], add=True)
        base = wid * EPW + c * C
        pltpu.sync_copy(eaf, expatt.at[pl.ds(base * H, C * H)])

    start(0, 0)
    start(1, 1)

    @pl.loop(0, NCHUNK, step=2)
    def _chunk(i):
        finish(0, i)

        @pl.when(i + 2 < NCHUNK)
        def _s0():
            start(0, i + 2)

        @pl.when(i + 1 < NCHUNK)
        def _odd():
            finish(1, i + 1)

            @pl.when(i + 3 < NCHUNK)
            def _s1():
                start(1, i + 3)

    plsc.subcore_barrier()
    pltpu.sync_copy(norm_sh.at[pl.ds(sid * STRIPE, STRIPE)], zbuf)
    rsub = lax.shift_right_logical(iota, 2)
    csub = lax.bitwise_and(iota, 3)

    @pl.loop(0, STRIPE, step=4)
    def _repack(r):
        v = plsc.load_gather(zbuf, [r + rsub, csub])
        bflat[pl.ds(r * H, L)] = v

    out0 = cid * (NPAD * H) + sid * (STRIPE * H)
    pltpu.sync_copy(bflat, normp.at[pl.ds(out0, STRIPE * H)])


def _edge_pass(qn, kn, rows, cols):
    f = pl.kernel(
        _edge_body,
        out_type=[
            jax.ShapeDtypeStruct((E * H,), jnp.float32),
            jax.ShapeDtypeStruct((NC * NPAD * H,), jnp.float32),
        ],
        mesh=_mesh(),
        scratch_types=[
            pltpu.VMEM((C,), jnp.int32),
            pltpu.VMEM((C,), jnp.int32),
            pltpu.VMEM((C, D), jnp.float32),
            pltpu.VMEM((C, D), jnp.float32),
            pltpu.VMEM((C,), jnp.int32),
            pltpu.VMEM((C,), jnp.int32),
            pltpu.VMEM((C, D), jnp.float32),
            pltpu.VMEM((C, D), jnp.float32),
            pltpu.VMEM((C, L), jnp.float32),
            pltpu.VMEM((C * H,), jnp.float32),
            pltpu.VMEM((STRIPE * H,), jnp.float32),
            pltpu.VMEM_SHARED((NPAD, L), jnp.float32),
            pltpu.VMEM((STRIPE, L), jnp.float32),
            pltpu.SemaphoreType.DMA,
            pltpu.SemaphoreType.DMA,
            pltpu.SemaphoreType.DMA,
            pltpu.SemaphoreType.DMA,
        ],
        compiler_params=_SC_PARAMS,
    )
    return f(qn, kn, rows, cols)


def _recip_body(n_ref, r_ref):
    half = NPAD * H // D
    r_ref[...] = 1.0 / (n_ref[:half] + n_ref[half:] + _EPS)


def _recip(normp):
    n2 = normp.reshape(NC * NPAD * H // D, D)
    out = pl.pallas_call(
        _recip_body,
        out_shape=jax.ShapeDtypeStruct((NPAD * H // D, D), jnp.float32),
    )(n2)
    return out.reshape(NPAD * H)


def _aggr_body(vn, rows, cols, expatt, recip, att, numerp,
               rv0, cv0, ve0, ea0, rv1, cv1, ve1, ea1,
               idx2, rcp, attb, val, numer_sh, zbuf,
               sv0, sv1, semr):
    cid = lax.axis_index("c")
    sid = lax.axis_index("s")
    wid = cid * NS + sid
    bufs = ((rv0, cv0, ve0, ea0, sv0), (rv1, cv1, ve1, ea1, sv1))

    zero16 = jnp.zeros((L,), jnp.float32)

    @pl.loop(0, STRIPE // 5)
    def _zero(r):
        for j in range(D // L):
            zbuf[r, pl.ds(L * j, L)] = zero16

    for p in range(5):
        pltpu.sync_copy(
            zbuf, numer_sh.at[pl.ds(sid * STRIPE + p * (STRIPE // 5),
                                    STRIPE // 5)])
    plsc.subcore_barrier()

    iota = lax.iota(jnp.int32, L)
    e_sub = lax.shift_right_logical(iota, 2)
    h_sub = lax.bitwise_and(iota, 3)

    def start(b, c):
        rv, cv, ve, ea, sv = bufs[b]
        base = wid * EPW + c * C
        pltpu.sync_copy(rows.at[pl.ds(base, C)], rv)
        pltpu.sync_copy(cols.at[pl.ds(base, C)], cv)
        pltpu.sync_copy(expatt.at[pl.ds(base * H, C * H)], ea)
        pltpu.async_copy(vn.at[cv], ve, sv)

    def finish(b, c):
        rv, cv, ve, ea, sv = bufs[b]

        @plsc.parallel_loop(0, C, step=4, unroll=4)
        def _idx16(e0):
            rvv = plsc.load_gather(rv, [e0 + e_sub])
            j = e0 // (C // H)
            u = e0 % (C // H)
            idx2[j, pl.ds(u * H, L)] = rvv * H + h_sub

        cps = [pltpu.async_copy(recip.at[idx2.at[j]],
                                rcp.at[pl.ds(C * j, C)], semr)
               for j in range(H)]
        for cp in cps:
            cp.wait()

        @plsc.parallel_loop(0, C * H // L, unroll=4)
        def _att16(u):
            attb[pl.ds(L * u, L)] = ea[pl.ds(L * u, L)] * rcp[pl.ds(L * u, L)]

        pltpu.make_async_copy(vn.at[cv], ve, sv).wait()

        @plsc.parallel_loop(0, C, unroll=4)
        def _edge(e):
            for h in range(H):
                sp = plsc.load_gather(attb, [jnp.full((L,), e * H + h,
                                                      jnp.int32)])
                o = 2 * h * L
                val[e, pl.ds(o, L)] = sp * ve[e, pl.ds(o, L)]
                val[e, pl.ds(o + L, L)] = sp * ve[e, pl.ds(o + L, L)]

        pltpu.sync_copy(val, numer_sh.at[rv], add=True)
        base = wid * EPW + c * C
        pltpu.sync_copy(attb, att.at[pl.ds(base * H, C * H)])

    start(0, 0)
    start(1, 1)

    @pl.loop(0, NCHUNK, step=2)
    def _chunk(i):
        finish(0, i)

        @pl.when(i + 2 < NCHUNK)
        def _s0():
            start(0, i + 2)

        @pl.when(i + 1 < NCHUNK)
        def _odd():
            finish(1, i + 1)

            @pl.when(i + 3 < NCHUNK)
            def _s1():
                start(1, i + 3)

    plsc.subcore_barrier()
    sl = pl.ds(sid * STRIPE, STRIPE)
    pltpu.sync_copy(numer_sh.at[sl], numerp.at[cid, sl])


def _aggr_pass(vn, rows, cols, expatt, recip):
    f = pl.kernel(
        _aggr_body,
        out_type=[
            jax.ShapeDtypeStruct((E * H,), jnp.float32),
            jax.ShapeDtypeStruct((NC, NPAD, D), jnp.float32),
        ],
        mesh=_mesh(),
        scratch_types=[
            pltpu.VMEM((C,), jnp.int32),
            pltpu.VMEM((C,), jnp.int32),
            pltpu.VMEM((C, D), jnp.float32),
            pltpu.VMEM((C * H,), jnp.float32),
            pltpu.VMEM((C,), jnp.int32),
            pltpu.VMEM((C,), jnp.int32),
            pltpu.VMEM((C, D), jnp.float32),
            pltpu.VMEM((C * H,), jnp.float32),
            pltpu.VMEM((H, C), jnp.int32),
            pltpu.VMEM((C * H,), jnp.float32),
            pltpu.VMEM((C * H,), jnp.float32),
            pltpu.VMEM((C, D), jnp.float32),
            pltpu.VMEM_SHARED((NPAD, D), jnp.float32),
            pltpu.VMEM((STRIPE // 5, D), jnp.float32),
            pltpu.SemaphoreType.DMA,
            pltpu.SemaphoreType.DMA,
            pltpu.SemaphoreType.DMA,
        ],
        compiler_params=_SC_PARAMS,
    )
    return f(vn, rows, cols, expatt, recip)


def _sum_body(n_ref, o_ref):
    o_ref[...] = n_ref[0] + n_ref[1]


def _psum(numerp):
    blk = 2048
    return pl.pallas_call(
        _sum_body,
        grid=(NPAD // blk,),
        in_specs=[pl.BlockSpec((NC, blk, D), lambda i: (0, i, 0))],
        out_specs=pl.BlockSpec((blk, D), lambda i: (i, 0)),
        out_shape=jax.ShapeDtypeStruct((NPAD, D), jnp.float32),
    )(numerp)


@jax.jit
def kernel(embeds, edge_index, qTrans, kTrans, vTrans):
    rows = edge_index[0, :].astype(jnp.int32)
    cols = edge_index[1, :].astype(jnp.int32)
    emb_pad = jnp.pad(embeds, ((0, NPAD - N), (0, 0)))

    qn, kn, vn = _qkv(emb_pad, qTrans, kTrans, vTrans)
    expatt, normp = _edge_pass(qn, kn, rows, cols)
    recip = _recip(normp)
    attf, numerp = _aggr_pass(vn, rows, cols, expatt, recip)
    res = _psum(numerp)[:N]
    att = attf.reshape(E, H)
    return res, att
